# baseline (speedup 1.0000x reference)
import functools

import jax
import jax.numpy as jnp
from jax import lax
from jax.experimental import pallas as pl
from jax.experimental.pallas import tpu as pltpu
from jax.experimental.pallas import tpu_sc as plsc

VOCAB_ROWS = 8021
DIM = 312
DIM_PAD = 384
TAIL = DIM - 256
NUM_IDX = 4096 * 200

NUM_CORES = 2
NUM_SUBCORES = 16
NUM_WORKERS = NUM_CORES * NUM_SUBCORES

CHUNK = 80
NBUF = 3
ROWS_PER_WORKER = NUM_IDX // NUM_WORKERS
CHUNKS_PER_WORKER = ROWS_PER_WORKER // CHUNK
LOOKAHEAD = 2

TAIL_OFFS = (0, 16, 32, TAIL - 16)


def _gather_body(table_hbm, idx_hbm, out_hbm, idx_bufs, rows, tails, isems,
                 gsems, wsems):
    wid = lax.axis_index("s") * NUM_CORES + lax.axis_index("c")
    row_base = wid * ROWS_PER_WORKER

    def start_idx(c, b):
        pltpu.async_copy(idx_hbm.at[pl.ds(row_base + c * CHUNK, CHUNK)],
                         idx_bufs.at[b], isems.at[b])

    def wait_idx(b):
        pltpu.make_async_copy(idx_hbm.at[pl.ds(row_base, CHUNK)],
                              idx_bufs.at[b], isems.at[b]).wait()

    def start_gather(b):
        pltpu.async_copy(table_hbm.at[idx_bufs.at[b]], rows.at[b], gsems.at[b])

    def wait_gather(b):
        pltpu.make_async_copy(table_hbm.at[idx_bufs.at[0]], rows.at[b],
                              gsems.at[b]).wait()

    def repack_tail(b):
        def row_body(r, carry):
            for o in TAIL_OFFS:
                tails[b, r, pl.ds(o, 16)] = rows[b, r, pl.ds(256 + o, 16)]
            return carry
        lax.fori_loop(0, CHUNK, row_body, 0)

    def start_write(c, b):
        dst = out_hbm.at[pl.ds(row_base + c * CHUNK, CHUNK)]
        pltpu.async_copy(rows.at[b, slice(None), pl.ds(0, 256)],
                         dst.at[slice(None), pl.ds(0, 256)], wsems.at[b])
        pltpu.async_copy(tails.at[b], dst.at[slice(None), pl.ds(256, TAIL)],
                         wsems.at[b])

    def wait_write(b):
        dst = out_hbm.at[pl.ds(row_base, CHUNK)]
        pltpu.make_async_copy(rows.at[b, slice(None), pl.ds(0, 256)],
                              dst.at[slice(None), pl.ds(0, 256)],
                              wsems.at[b]).wait()
        pltpu.make_async_copy(tails.at[b],
                              dst.at[slice(None), pl.ds(256, TAIL)],
                              wsems.at[b]).wait()

    for c in range(LOOKAHEAD):
        start_idx(c, c % NBUF)
    for c in range(LOOKAHEAD):
        wait_idx(c % NBUF)
        start_gather(c % NBUF)

    def step(c, b, bf):
        cf = c + LOOKAHEAD

        @pl.when(cf < CHUNKS_PER_WORKER)
        def _():
            @pl.when(cf >= NBUF)
            def _():
                wait_write(bf)
            start_idx(cf, bf)
            wait_idx(bf)
            start_gather(bf)

        wait_gather(b)
        repack_tail(b)
        start_write(c, b)

    def step3(s, carry):
        for i in range(NBUF):
            step(s * NBUF + i, i, (i + LOOKAHEAD) % NBUF)
        return carry

    n_full = CHUNKS_PER_WORKER // NBUF
    lax.fori_loop(0, n_full, step3, 0)
    for c in range(n_full * NBUF, CHUNKS_PER_WORKER):
        step(c, c % NBUF, (c + LOOKAHEAD) % NBUF)

    for b in range(NBUF):
        wait_write(b)


@jax.jit
def _embedding_gather(table, idx):
    mesh = plsc.VectorSubcoreMesh(core_axis_name="c", subcore_axis_name="s")
    run = functools.partial(
        pl.kernel,
        out_type=jax.ShapeDtypeStruct((NUM_IDX, DIM), jnp.float32),
        mesh=mesh,
        scratch_types=[
            pltpu.VMEM((NBUF, CHUNK), jnp.int32),
            pltpu.VMEM((NBUF, CHUNK, DIM_PAD), jnp.float32),
            pltpu.VMEM((NBUF, CHUNK, TAIL), jnp.float32),
            pltpu.SemaphoreType.DMA((NBUF,)),
            pltpu.SemaphoreType.DMA((NBUF,)),
            pltpu.SemaphoreType.DMA((NBUF,)),
        ],
        compiler_params=pltpu.CompilerParams(use_tc_tiling_on_sc=True),
    )(_gather_body)
    return run(table, idx)


def kernel(x, mask, emb):
    idx = x.reshape(-1).astype(jnp.int32)
    table = jnp.pad(emb, ((0, 0), (0, DIM_PAD - DIM)))
    flat = _embedding_gather(table, idx)
    y = flat.reshape(x.shape[0], x.shape[1], DIM)
    return (y, y)


# speedup vs baseline: 1.3656x; 1.3656x over previous
import functools

import jax
import jax.numpy as jnp
from jax import lax
from jax.experimental import layout as jax_layout
from jax.experimental import pallas as pl
from jax.experimental.pallas import tpu as pltpu
from jax.experimental.pallas import tpu_sc as plsc

VOCAB_ROWS = 8021
DIM = 312
DIM_PAD = 384
TAIL = DIM - 256
NUM_IDX = 4096 * 200

NUM_CORES = 2
NUM_SUBCORES = 16
NUM_WORKERS = NUM_CORES * NUM_SUBCORES

CHUNK = 80
NBUF = 3
ROWS_PER_WORKER = NUM_IDX // NUM_WORKERS
CHUNKS_PER_WORKER = ROWS_PER_WORKER // CHUNK
LOOKAHEAD = 2

TAIL_OFFS = (0, 16, 32, TAIL - 16)


def _gather_body(table_hbm, idx_hbm, out_hbm, idx_bufs, rows, tails, isems,
                 gsems, wsems):
    wid = lax.axis_index("s") * NUM_CORES + lax.axis_index("c")
    row_base = wid * ROWS_PER_WORKER

    def start_idx(c, b):
        pltpu.async_copy(idx_hbm.at[pl.ds(row_base + c * CHUNK, CHUNK)],
                         idx_bufs.at[b], isems.at[b])

    def wait_idx(b):
        pltpu.make_async_copy(idx_hbm.at[pl.ds(row_base, CHUNK)],
                              idx_bufs.at[b], isems.at[b]).wait()

    def start_gather(b):
        pltpu.async_copy(table_hbm.at[idx_bufs.at[b]], rows.at[b], gsems.at[b])

    def wait_gather(b):
        pltpu.make_async_copy(table_hbm.at[idx_bufs.at[0]], rows.at[b],
                              gsems.at[b]).wait()

    def repack_tail(b):
        def row_body(r, carry):
            for o in TAIL_OFFS:
                tails[b, r, pl.ds(o, 16)] = rows[b, r, pl.ds(256 + o, 16)]
            return carry
        lax.fori_loop(0, CHUNK, row_body, 0)

    def start_write(c, b):
        dst = out_hbm.at[pl.ds(row_base + c * CHUNK, CHUNK)]
        pltpu.async_copy(rows.at[b, slice(None), pl.ds(0, 256)],
                         dst.at[slice(None), pl.ds(0, 256)], wsems.at[b])
        pltpu.async_copy(tails.at[b], dst.at[slice(None), pl.ds(256, TAIL)],
                         wsems.at[b])

    def wait_write(b):
        dst = out_hbm.at[pl.ds(row_base, CHUNK)]
        pltpu.make_async_copy(rows.at[b, slice(None), pl.ds(0, 256)],
                              dst.at[slice(None), pl.ds(0, 256)],
                              wsems.at[b]).wait()
        pltpu.make_async_copy(tails.at[b],
                              dst.at[slice(None), pl.ds(256, TAIL)],
                              wsems.at[b]).wait()

    for c in range(LOOKAHEAD):
        start_idx(c, c % NBUF)
    for c in range(LOOKAHEAD):
        wait_idx(c % NBUF)
        start_gather(c % NBUF)

    def step(c, b, bf):
        cf = c + LOOKAHEAD

        @pl.when(cf < CHUNKS_PER_WORKER)
        def _():
            @pl.when(cf >= NBUF)
            def _():
                wait_write(bf)
            start_idx(cf, bf)
            wait_idx(bf)
            start_gather(bf)

        wait_gather(b)
        repack_tail(b)
        start_write(c, b)

    def step3(s, carry):
        for i in range(NBUF):
            step(s * NBUF + i, i, (i + LOOKAHEAD) % NBUF)
        return carry

    n_full = CHUNKS_PER_WORKER // NBUF
    lax.fori_loop(0, n_full, step3, 0)
    for c in range(n_full * NBUF, CHUNKS_PER_WORKER):
        step(c, c % NBUF, (c + LOOKAHEAD) % NBUF)

    for b in range(NBUF):
        wait_write(b)


@jax.jit
def _embedding_gather(table, idx):
    mesh = plsc.VectorSubcoreMesh(core_axis_name="c", subcore_axis_name="s")
    run = functools.partial(
        pl.kernel,
        out_type=jax.ShapeDtypeStruct((NUM_IDX, DIM), jnp.float32),
        mesh=mesh,
        scratch_types=[
            pltpu.VMEM((NBUF, CHUNK), jnp.int32),
            pltpu.VMEM((NBUF, CHUNK, DIM_PAD), jnp.float32),
            pltpu.VMEM((NBUF, CHUNK, TAIL), jnp.float32),
            pltpu.SemaphoreType.DMA((NBUF,)),
            pltpu.SemaphoreType.DMA((NBUF,)),
            pltpu.SemaphoreType.DMA((NBUF,)),
        ],
        compiler_params=pltpu.CompilerParams(use_tc_tiling_on_sc=True),
    )(_gather_body)
    return run(table, idx)


def kernel(x, mask, emb):
    idx = x.reshape(-1).astype(jnp.int32)
    table = jnp.pad(emb, ((0, 0), (0, DIM_PAD - DIM)))
    flat = _embedding_gather(table, idx)
    y = flat.reshape(x.shape[0], x.shape[1], DIM)
    y = jax_layout.with_layout_constraint(
        y, jax_layout.Layout(major_to_minor=(0, 1, 2)))
    return (y, y)
